# baseline (speedup 1.0000x reference)
import functools

import jax
import jax.numpy as jnp
from jax import lax
from jax.experimental import pallas as pl
from jax.experimental.pallas import tpu as pltpu
from jax.experimental.pallas import tpu_sc as plsc

N_LAT = 32
N_OUT = 32
LANES = 16


def _build(B, N_SAMPLE):
    info = plsc.get_sparse_core_info()
    NC, NS = info.num_cores, info.num_subcores
    NW = NC * NS
    assert B % NW == 0
    RPW = B // NW
    C = 8
    NCHUNK = RPW // C

    mesh = plsc.VectorSubcoreMesh(core_axis_name="c", subcore_axis_name="s")

    @functools.partial(
        pl.kernel,
        mesh=mesh,
        out_type=jax.ShapeDtypeStruct((B, N_OUT), jnp.float32),
        compiler_params=pltpu.CompilerParams(use_tc_tiling_on_sc=True),
        scratch_types=[
            pltpu.VMEM((RPW,), jnp.int32),
            pltpu.VMEM((C, 8, 128), jnp.float32),
            pltpu.VMEM((C, 128), jnp.float32),
            pltpu.VMEM((C, N_LAT), jnp.float32),
            pltpu.VMEM((C, N_OUT), jnp.float32),
            pltpu.SemaphoreType.DMA,
            pltpu.SemaphoreType.DMA,
        ],
    )
    def k(u_hbm, si_hbm, amat_hbm, offs_hbm, out_hbm,
          idx_v, as_v, og_v, u_v, out_v, sem_a, sem_o):
        wid = lax.axis_index("s") * NC + lax.axis_index("c")
        base = wid * RPW
        pltpu.sync_copy(si_hbm.at[pl.ds(base, RPW)], idx_v)

        def chunk(g, carry):
            off = g * C
            pltpu.async_copy(amat_hbm.at[idx_v.at[pl.ds(off, C)]], as_v, sem_a).wait()
            pltpu.async_copy(offs_hbm.at[idx_v.at[pl.ds(off, C)]], og_v, sem_o).wait()
            pltpu.sync_copy(u_hbm.at[pl.ds(base + off, C)], u_v)
            for r in range(C):
                uv0 = u_v[r, pl.ds(0, LANES)]
                uv1 = u_v[r, pl.ds(LANES, LANES)]
                acc0 = uv0 + og_v[r, pl.ds(0, LANES)]
                acc1 = uv1 + og_v[r, pl.ds(LANES, LANES)]
                for l in range(N_LAT):
                    ul = (uv0 if l < LANES else uv1)[l % LANES]
                    acc0 = acc0 + ul * as_v[r, l // 4, pl.ds((l % 4) * N_OUT, LANES)]
                    acc1 = acc1 + ul * as_v[r, l // 4, pl.ds((l % 4) * N_OUT + LANES, LANES)]
                out_v[r, pl.ds(0, LANES)] = acc0
                out_v[r, pl.ds(LANES, LANES)] = acc1
            pltpu.sync_copy(out_v, out_hbm.at[pl.ds(base + off, C)])
            return carry

        lax.fori_loop(0, NCHUNK, chunk, 0)

    return k


def kernel(u, sample_index, amat_sample, offsets):
    B = u.shape[0]
    n_sample = amat_sample.shape[0]
    si = sample_index.squeeze() if sample_index.ndim > 1 else sample_index
    offs_pad = jnp.pad(offsets, ((0, 0), (0, 128 - N_OUT)))
    amat3 = amat_sample.reshape(n_sample, 8, 128)
    k = _build(B, n_sample)
    return k(u, si.astype(jnp.int32), amat3, offs_pad)


# speedup vs baseline: 1.3102x; 1.3102x over previous
import functools

import jax
import jax.numpy as jnp
from jax import lax
from jax.experimental import pallas as pl
from jax.experimental.pallas import tpu as pltpu
from jax.experimental.pallas import tpu_sc as plsc

N_LAT = 32
N_OUT = 32
LANES = 16


def _build(B, N_SAMPLE):
    info = plsc.get_sparse_core_info()
    NC, NS = info.num_cores, info.num_subcores
    NW = NC * NS
    assert B % NW == 0
    RPW = B // NW
    C = 32
    NPAIR = RPW // (2 * C)

    mesh = plsc.VectorSubcoreMesh(core_axis_name="c", subcore_axis_name="s")

    @functools.partial(
        pl.kernel,
        mesh=mesh,
        out_type=jax.ShapeDtypeStruct((N_OUT, B), jnp.float32),
        compiler_params=pltpu.CompilerParams(needs_layout_passes=False),
        scratch_types=[
            pltpu.VMEM((RPW,), jnp.int32),
            pltpu.VMEM((C, 8, 128), jnp.float32),
            pltpu.VMEM((C, 8, 128), jnp.float32),
            pltpu.VMEM((C, 128), jnp.float32),
            pltpu.VMEM((C, 128), jnp.float32),
            pltpu.VMEM((N_LAT, RPW), jnp.float32),
            pltpu.VMEM((N_OUT, RPW), jnp.float32),
            pltpu.SemaphoreType.DMA,
            pltpu.SemaphoreType.DMA,
            pltpu.SemaphoreType.DMA,
            pltpu.SemaphoreType.DMA,
        ],
    )
    def k(u_t_hbm, si_hbm, amat_hbm, offs_hbm, out_t_hbm,
          idx_v, as_p, as_q, og_p, og_q, u_slab, out_slab,
          sem_as_p, sem_as_q, sem_og_p, sem_og_q):
        wid = lax.axis_index("s") * NC + lax.axis_index("c")
        base = wid * RPW
        pltpu.sync_copy(si_hbm.at[pl.ds(base, RPW)], idx_v)
        pltpu.sync_copy(u_t_hbm.at[:, pl.ds(base, RPW)], u_slab)

        def start(off, as_b, og_b, sem_a, sem_o):
            pltpu.async_copy(amat_hbm.at[idx_v.at[pl.ds(off, C)]], as_b, sem_a)
            pltpu.async_copy(offs_hbm.at[idx_v.at[pl.ds(off, C)]], og_b, sem_o)

        def wait(as_b, og_b, sem_a, sem_o):
            pltpu.make_async_copy(amat_hbm.at[idx_v.at[pl.ds(0, C)]], as_b, sem_a).wait()
            pltpu.make_async_copy(offs_hbm.at[idx_v.at[pl.ds(0, C)]], og_b, sem_o).wait()

        def compute(off, as_b, og_b):
            def octet(ro, carry):
                rbase = ro * 8
                for rr in range(8):
                    r = rbase + rr
                    j = off + r
                    rows = lax.iota(jnp.int32, LANES)
                    cols = jnp.full((LANES,), j, jnp.int32)
                    uv0 = plsc.load_gather(u_slab, [rows, cols])
                    uv1 = plsc.load_gather(u_slab, [rows + LANES, cols])
                    acc0 = uv0 + og_b[r, pl.ds(0, LANES)]
                    acc1 = uv1 + og_b[r, pl.ds(LANES, LANES)]
                    for l in range(N_LAT):
                        ul = (uv0 if l < LANES else uv1)[l % LANES]
                        acc0 = acc0 + ul * as_b[r, l // 4, pl.ds((l % 4) * N_OUT, LANES)]
                        acc1 = acc1 + ul * as_b[r, l // 4, pl.ds((l % 4) * N_OUT + LANES, LANES)]
                    plsc.store_scatter(out_slab, [rows, cols], acc0)
                    plsc.store_scatter(out_slab, [rows + LANES, cols], acc1)
                return carry

            lax.fori_loop(0, C // 8, octet, 0)

        start(0, as_p, og_p, sem_as_p, sem_og_p)

        def pair(i, carry):
            offp = (2 * i) * C
            offq = (2 * i + 1) * C
            start(offq, as_q, og_q, sem_as_q, sem_og_q)
            wait(as_p, og_p, sem_as_p, sem_og_p)
            compute(offp, as_p, og_p)

            @pl.when(i < NPAIR - 1)
            def _():
                start(offq + C, as_p, og_p, sem_as_p, sem_og_p)

            wait(as_q, og_q, sem_as_q, sem_og_q)
            compute(offq, as_q, og_q)
            return carry

        lax.fori_loop(0, NPAIR, pair, 0)
        pltpu.sync_copy(out_slab, out_t_hbm.at[:, pl.ds(base, RPW)])

    return k


def kernel(u, sample_index, amat_sample, offsets):
    B = u.shape[0]
    n_sample = amat_sample.shape[0]
    si = sample_index.squeeze() if sample_index.ndim > 1 else sample_index
    offs_pad = jnp.pad(offsets, ((0, 0), (0, 128 - N_OUT)))
    amat3 = amat_sample.reshape(n_sample, 8, 128)
    k = _build(B, n_sample)
    out_t = k(u.T, si.astype(jnp.int32), amat3, offs_pad)
    return out_t.T
